# Initial kernel scaffold; baseline (speedup 1.0000x reference)
import functools

import jax
import jax.numpy as jnp
from jax import lax
from jax.experimental import pallas as pl
from jax.experimental.pallas import tpu as pltpu
from jax.experimental.pallas import tpu_sc as plsc

_N = 10000
_E = 320000
_D = 128
_H = 64
_NS = 16
_K = 80
_EPT = _E // _NS
_NCHUNK = _EPT // _K
_RPT = _N // _NS
_ZR = 125
_RB = 2000

_HIGH = lax.Precision.HIGHEST


def _dot(a, b):
    return jnp.dot(a, b, preferred_element_type=jnp.float32, precision=_HIGH)


def _tc_in_body(x_ref, win_ref, bin_ref, cw_ref, cb_ref, ew_ref, eb_ref,
                h_ref, e_ref):
    x1 = _dot(x_ref[...], win_ref[...]) + bin_ref[...]
    e_ref[...] = _dot(x1, ew_ref[...]) + eb_ref[...]
    h = _dot(x1, cw_ref[...]) + cb_ref[...]
    h_ref[0] = h[:, :_H]
    h_ref[1] = h[:, _H:]


def _tc_in(x, W_in, b_in, cW, cb, eWt, ebt):
    return pl.pallas_call(
        _tc_in_body,
        grid=(_N // _RB,),
        in_specs=[
            pl.BlockSpec((_RB, _D), lambda g: (g, 0)),
            pl.BlockSpec((_D, _D), lambda g: (0, 0)),
            pl.BlockSpec((1, _D), lambda g: (0, 0)),
            pl.BlockSpec((_D, _D), lambda g: (0, 0)),
            pl.BlockSpec((1, _D), lambda g: (0, 0)),
            pl.BlockSpec((_D, 1), lambda g: (0, 0)),
            pl.BlockSpec((1, 1), lambda g: (0, 0)),
        ],
        out_specs=[
            pl.BlockSpec((2, _RB, _H), lambda g: (0, g, 0)),
            pl.BlockSpec((_RB, 1), lambda g: (g, 0)),
        ],
        out_shape=[
            jax.ShapeDtypeStruct((2, _N, _H), jnp.float32),
            jax.ShapeDtypeStruct((_N, 1), jnp.float32),
        ],
    )(x, W_in, b_in, cW, cb, eWt, ebt)


def _tc_mid_body(a0_ref, a1_ref, ep_ref, cw_ref, cb_ref, ew_ref, eb_ref,
                 h_ref, e_ref):
    xa = jnp.concatenate([a0_ref[...], a1_ref[...]], axis=1)
    x2 = jnp.where(xa > 0, xa, 0.01 * xa)
    e_ref[...] = ep_ref[...] + _dot(x2, ew_ref[...]) + eb_ref[...]
    h = _dot(x2, cw_ref[...]) + cb_ref[...]
    h_ref[0] = h[:, :_H]
    h_ref[1] = h[:, _H:]


def _tc_mid(agg, e_prev, cW, cb, eWt, ebt):
    return pl.pallas_call(
        _tc_mid_body,
        grid=(_N // _RB,),
        in_specs=[
            pl.BlockSpec((_RB, _H), lambda g: (g, 0)),
            pl.BlockSpec((_RB, _H), lambda g: (g + _N // _RB, 0)),
            pl.BlockSpec((_RB, 1), lambda g: (g, 0)),
            pl.BlockSpec((_D, _D), lambda g: (0, 0)),
            pl.BlockSpec((1, _D), lambda g: (0, 0)),
            pl.BlockSpec((_D, 1), lambda g: (0, 0)),
            pl.BlockSpec((1, 1), lambda g: (0, 0)),
        ],
        out_specs=[
            pl.BlockSpec((2, _RB, _H), lambda g: (0, g, 0)),
            pl.BlockSpec((_RB, 1), lambda g: (g, 0)),
        ],
        out_shape=[
            jax.ShapeDtypeStruct((2, _N, _H), jnp.float32),
            jax.ShapeDtypeStruct((_N, 1), jnp.float32),
        ],
    )(agg, agg, e_prev, cW, cb, eWt, ebt)


def _tc_out_body(a0_ref, a1_ref, ep_ref, ew_ref, eb_ref, e_ref):
    xa = jnp.concatenate([a0_ref[...], a1_ref[...]], axis=1)
    x3 = jnp.where(xa > 0, xa, 0.01 * xa)
    e_ref[...] = ep_ref[...] + _dot(x3, ew_ref[...]) + eb_ref[...]


def _tc_out(agg, e_prev, eWt, ebt):
    return pl.pallas_call(
        _tc_out_body,
        grid=(_N // _RB,),
        in_specs=[
            pl.BlockSpec((_RB, _H), lambda g: (g, 0)),
            pl.BlockSpec((_RB, _H), lambda g: (g + _N // _RB, 0)),
            pl.BlockSpec((_RB, 1), lambda g: (g, 0)),
            pl.BlockSpec((_D, 1), lambda g: (0, 0)),
            pl.BlockSpec((1, 1), lambda g: (0, 0)),
        ],
        out_specs=pl.BlockSpec((_RB, 1), lambda g: (g, 0)),
        out_shape=jax.ShapeDtypeStruct((_N, 1), jnp.float32),
    )(agg, agg, e_prev, eWt, ebt)


@functools.partial(
    pl.kernel,
    out_type=jax.ShapeDtypeStruct((2 * _N, _H), jnp.float32),
    mesh=plsc.VectorSubcoreMesh(core_axis_name="c", subcore_axis_name="s"),
    scratch_types=[
        pltpu.VMEM_SHARED((_N, _H), jnp.float32),
        pltpu.VMEM((_NCHUNK, _K), jnp.int32),
        pltpu.VMEM((_NCHUNK, _K), jnp.int32),
        pltpu.VMEM((_NCHUNK, _K), jnp.float32),
        pltpu.VMEM((_K, _H), jnp.float32),
        pltpu.VMEM((_ZR, _H), jnp.float32),
        pltpu.SemaphoreType.DMA,
    ],
)
def _sc_sweep(h_hbm, srcp_hbm, dst_hbm, w_hbm, out_hbm,
              agg_sh, src_v, dst_v, w_v, rows_v, zbuf, sem):
    cid = lax.axis_index("c")
    sid = lax.axis_index("s")

    def _z(r, _):
        for c in range(_H // 16):
            zbuf[r, pl.ds(c * 16, 16)] = jnp.zeros((16,), jnp.float32)
        return 0
    lax.fori_loop(0, _ZR, _z, 0)
    for j in range(_RPT // _ZR):
        pltpu.sync_copy(zbuf, agg_sh.at[pl.ds(sid * _RPT + j * _ZR, _ZR)])
    plsc.subcore_barrier()

    row0 = sid * _NCHUNK
    pltpu.sync_copy(srcp_hbm.at[cid, pl.ds(row0, _NCHUNK)], src_v)
    pltpu.sync_copy(dst_hbm.at[pl.ds(row0, _NCHUNK)], dst_v)
    pltpu.sync_copy(w_hbm.at[pl.ds(row0, _NCHUNK)], w_v)

    def _chunk(g, _):
        pltpu.async_copy(h_hbm.at[src_v.at[g]], rows_v, sem).wait()

        def _scale(e, _2):
            w = w_v[g, e]
            for c in range(_H // 16):
                sl = pl.ds(c * 16, 16)
                rows_v[e, sl] = rows_v[e, sl] * w
            return 0
        lax.fori_loop(0, _K, _scale, 0)
        pltpu.sync_copy(rows_v, agg_sh.at[dst_v.at[g]], add=True)
        return 0
    lax.fori_loop(0, _NCHUNK, _chunk, 0)
    plsc.subcore_barrier()

    pltpu.sync_copy(agg_sh.at[pl.ds(sid * _RPT, _RPT)],
                    out_hbm.at[pl.ds(cid * _N + sid * _RPT, _RPT)])


def kernel(x, edge_index, edge_weight, W_in, b_in, conv_W, conv_b,
           energy_W, energy_b, temp):
    src = edge_index[0]
    dst = edge_index[1]
    srcp = jnp.stack([src, src + _N]).reshape(2, _E // _K, _K)
    dst2 = dst.reshape(_E // _K, _K)
    w2 = edge_weight.reshape(_E // _K, _K)
    eWt = energy_W * temp[:, None, None]
    ebt = (energy_b * temp[:, None]).reshape(-1, 1, 1)
    b_in2 = b_in.reshape(1, _D)
    cb2 = conv_b.reshape(-1, 1, _D)

    h1, e0 = _tc_in(x, W_in, b_in2, conv_W[0], cb2[0], eWt[0], ebt[0])
    agg1 = _sc_sweep(h1.reshape(2 * _N, _H), srcp, dst2, w2)
    h2, e01 = _tc_mid(agg1, e0, conv_W[1], cb2[1], eWt[1], ebt[1])
    agg2 = _sc_sweep(h2.reshape(2 * _N, _H), srcp, dst2, w2)
    return _tc_out(agg2, e01, eWt[2], ebt[2])


# speedup vs baseline: 5.5474x; 5.5474x over previous
import functools

import jax
import jax.numpy as jnp
from jax import lax
from jax.experimental import pallas as pl
from jax.experimental.pallas import tpu as pltpu
from jax.experimental.pallas import tpu_sc as plsc

_N = 10000
_E = 320000
_D = 128
_NS = 16
_K = 80
_EPT = _E // (2 * _NS)
_NSLAB = 5
_NCHUNK = _EPT // (_K * _NSLAB)
_NP = 10240
_RPT = _NP // _NS
_RB = 2000

_HIGH = lax.Precision.HIGHEST


def _dot(a, b):
    return jnp.dot(a, b, preferred_element_type=jnp.float32, precision=_HIGH)


def _tc_in_body(x_ref, win_ref, bin_ref, cw_ref, cb_ref, ew_ref, eb_ref,
                h_ref, e_ref):
    x1 = _dot(x_ref[...], win_ref[...]) + bin_ref[...]
    e_ref[...] = _dot(x1, ew_ref[...]) + eb_ref[...]
    h_ref[...] = _dot(x1, cw_ref[...]) + cb_ref[...]


def _tc_in(x, W_in, b_in, cW, cb, eWt, ebt):
    return pl.pallas_call(
        _tc_in_body,
        grid=(_N // _RB,),
        in_specs=[
            pl.BlockSpec((_RB, _D), lambda g: (g, 0)),
            pl.BlockSpec((_D, _D), lambda g: (0, 0)),
            pl.BlockSpec((1, _D), lambda g: (0, 0)),
            pl.BlockSpec((_D, _D), lambda g: (0, 0)),
            pl.BlockSpec((1, _D), lambda g: (0, 0)),
            pl.BlockSpec((_D, 1), lambda g: (0, 0)),
            pl.BlockSpec((1, 1), lambda g: (0, 0)),
        ],
        out_specs=[
            pl.BlockSpec((_RB, _D), lambda g: (g, 0)),
            pl.BlockSpec((_RB, 1), lambda g: (g, 0)),
        ],
        out_shape=[
            jax.ShapeDtypeStruct((_NP, _D), jnp.float32),
            jax.ShapeDtypeStruct((_N, 1), jnp.float32),
        ],
    )(x, W_in, b_in, cW, cb, eWt, ebt)


def _tc_mid_body(a0_ref, a1_ref, ep_ref, cw_ref, cb_ref, ew_ref, eb_ref,
                 h_ref, e_ref):
    xa = a0_ref[0] + a1_ref[0]
    x2 = jnp.where(xa > 0, xa, 0.01 * xa)
    e_ref[...] = ep_ref[...] + _dot(x2, ew_ref[...]) + eb_ref[...]
    h_ref[...] = _dot(x2, cw_ref[...]) + cb_ref[...]


def _tc_mid(agg, e_prev, cW, cb, eWt, ebt):
    return pl.pallas_call(
        _tc_mid_body,
        grid=(_N // _RB,),
        in_specs=[
            pl.BlockSpec((1, _RB, _D), lambda g: (0, g, 0)),
            pl.BlockSpec((1, _RB, _D), lambda g: (1, g, 0)),
            pl.BlockSpec((_RB, 1), lambda g: (g, 0)),
            pl.BlockSpec((_D, _D), lambda g: (0, 0)),
            pl.BlockSpec((1, _D), lambda g: (0, 0)),
            pl.BlockSpec((_D, 1), lambda g: (0, 0)),
            pl.BlockSpec((1, 1), lambda g: (0, 0)),
        ],
        out_specs=[
            pl.BlockSpec((_RB, _D), lambda g: (g, 0)),
            pl.BlockSpec((_RB, 1), lambda g: (g, 0)),
        ],
        out_shape=[
            jax.ShapeDtypeStruct((_NP, _D), jnp.float32),
            jax.ShapeDtypeStruct((_N, 1), jnp.float32),
        ],
    )(agg, agg, e_prev, cW, cb, eWt, ebt)


def _tc_out_body(a0_ref, a1_ref, ep_ref, ew_ref, eb_ref, e_ref):
    xa = a0_ref[0] + a1_ref[0]
    x3 = jnp.where(xa > 0, xa, 0.01 * xa)
    e_ref[...] = ep_ref[...] + _dot(x3, ew_ref[...]) + eb_ref[...]


def _tc_out(agg, e_prev, eWt, ebt):
    return pl.pallas_call(
        _tc_out_body,
        grid=(_N // _RB,),
        in_specs=[
            pl.BlockSpec((1, _RB, _D), lambda g: (0, g, 0)),
            pl.BlockSpec((1, _RB, _D), lambda g: (1, g, 0)),
            pl.BlockSpec((_RB, 1), lambda g: (g, 0)),
            pl.BlockSpec((_D, 1), lambda g: (0, 0)),
            pl.BlockSpec((1, 1), lambda g: (0, 0)),
        ],
        out_specs=pl.BlockSpec((_RB, 1), lambda g: (g, 0)),
        out_shape=jax.ShapeDtypeStruct((_N, 1), jnp.float32),
    )(agg, agg, e_prev, eWt, ebt)


@functools.partial(
    pl.kernel,
    out_type=jax.ShapeDtypeStruct((2, _NP, _D), jnp.float32),
    mesh=plsc.VectorSubcoreMesh(core_axis_name="c", subcore_axis_name="s"),
    scratch_types=[
        pltpu.VMEM_SHARED((_NP, _D), jnp.float32),
        pltpu.VMEM((_NCHUNK, _K), jnp.int32),
        pltpu.VMEM((_NCHUNK, _K), jnp.int32),
        pltpu.VMEM((_NCHUNK, _K), jnp.float32),
        pltpu.VMEM((_K, _D), jnp.float32),
        pltpu.SemaphoreType.DMA,
    ],
)
def _sc_sweep(h_hbm, src_hbm, dst_hbm, w_hbm, out_hbm,
              agg_sh, src_v, dst_v, w_v, rows_v, sem):
    cid = lax.axis_index("c")
    sid = lax.axis_index("s")

    def _z(r, _):
        for c in range(_D // 16):
            rows_v[r, pl.ds(c * 16, 16)] = jnp.zeros((16,), jnp.float32)
        return 0
    lax.fori_loop(0, _K, _z, 0)
    for j in range(_RPT // _K):
        pltpu.sync_copy(rows_v, agg_sh.at[pl.ds(sid * _RPT + j * _K, _K)])
    plsc.subcore_barrier()

    def _slab(s, _0):
        pltpu.sync_copy(src_hbm.at[cid, sid, s], src_v)
        pltpu.sync_copy(dst_hbm.at[cid, sid, s], dst_v)
        pltpu.sync_copy(w_hbm.at[cid, sid, s], w_v)

        def _chunk(g, _):
            pltpu.async_copy(h_hbm.at[src_v.at[g]], rows_v, sem).wait()

            def _scale(b, _2):
                w16 = w_v[g, pl.ds(b * 16, 16)]
                for j in range(16):
                    e = b * 16 + j
                    w = w16[j]
                    for c in range(_D // 16):
                        sl = pl.ds(c * 16, 16)
                        rows_v[e, sl] = rows_v[e, sl] * w
                return 0
            lax.fori_loop(0, _K // 16, _scale, 0)
            pltpu.sync_copy(rows_v, agg_sh.at[dst_v.at[g]], add=True)
            return 0
        lax.fori_loop(0, _NCHUNK, _chunk, 0)
        return 0
    lax.fori_loop(0, _NSLAB, _slab, 0)
    plsc.subcore_barrier()

    pltpu.sync_copy(agg_sh.at[pl.ds(sid * _RPT, _RPT)],
                    out_hbm.at[cid, pl.ds(sid * _RPT, _RPT)])


def kernel(x, edge_index, edge_weight, W_in, b_in, conv_W, conv_b,
           energy_W, energy_b, temp):
    src2 = edge_index[0].reshape(2, _NS, _NSLAB, _NCHUNK, _K)
    dst2 = edge_index[1].reshape(2, _NS, _NSLAB, _NCHUNK, _K)
    w2 = edge_weight.reshape(2, _NS, _NSLAB, _NCHUNK, _K)
    eWt = energy_W * temp[:, None, None]
    ebt = (energy_b * temp[:, None]).reshape(-1, 1, 1)
    b_in2 = b_in.reshape(1, _D)
    cb2 = conv_b.reshape(-1, 1, _D)

    h1, e0 = _tc_in(x, W_in, b_in2, conv_W[0], cb2[0], eWt[0], ebt[0])
    agg1 = _sc_sweep(h1, src2, dst2, w2)
    h2, e01 = _tc_mid(agg1, e0, conv_W[1], cb2[1], eWt[1], ebt[1])
    agg2 = _sc_sweep(h2, src2, dst2, w2)
    return _tc_out(agg2, e01, eWt[2], ebt[2])
